# baseline (speedup 1.0000x reference)
import functools

import jax
import jax.numpy as jnp
from jax import lax
from jax.experimental import pallas as pl
from jax.experimental.pallas import tpu as pltpu
from jax.experimental.pallas import tpu_sc as plsc

N = 10000
E = 320000
D = 128
V = 128
G = 256
NUM_LAYERS = 3

NC = 2
NS = 16
NW = NC * NS
K = 128
NCHUNK = 80
E_PAD = NW * NCHUNK * K
EW = E_PAD // NW
STAGE = 40
NSTAGE = NCHUNK // STAGE
NCHUNKS_ALL = E_PAD // K
C_CORE0 = 40
C_CORE1 = 120
OFF_CORE1 = NS * C_CORE0
N_PAD = 10240
ROWS_PER_TILE = N_PAD // NS
FLUSH_CHUNKS = ROWS_PER_TILE // K
PAD_DST = N + 8

TB = 400
NTB = N // TB


def _make_segsum():
    mesh = plsc.VectorSubcoreMesh(core_axis_name="c", subcore_axis_name="s",
                                  num_cores=NC, num_subcores=NS)

    @functools.partial(
        pl.kernel,
        out_type=jax.ShapeDtypeStruct((NC, N_PAD, D), jnp.float32),
        mesh=mesh,
        scratch_types=[
            pltpu.VMEM((STAGE, K), jnp.int32),
            pltpu.VMEM((STAGE, K), jnp.int32),
            pltpu.VMEM((K, D), jnp.float32),
            pltpu.VMEM((K, D), jnp.float32),
            pltpu.VMEM_SHARED((N_PAD, D), jnp.float32),
            pltpu.SemaphoreType.DMA,
            pltpu.SemaphoreType.DMA,
        ],
    )
    def segsum(h_hbm, src_hbm, dst_hbm, out_hbm,
               src_v, dst_v, rows0_v, rows1_v, agg_sh, gsem, ssem):
        c = lax.axis_index("c")
        s = lax.axis_index("s")
        chunk_base = jnp.where(c == 0, s * C_CORE0, OFF_CORE1 + s * C_CORE1)
        nstages = jnp.where(c == 0, C_CORE0 // STAGE, C_CORE1 // STAGE)

        def zrow(i, _):
            for j in range(D // 16):
                rows0_v[i, pl.ds(j * 16, 16)] = jnp.zeros((16,), jnp.float32)
            return 0
        lax.fori_loop(0, K, zrow, 0)
        for f in range(FLUSH_CHUNKS):
            pltpu.sync_copy(rows0_v, agg_sh.at[pl.ds(s * ROWS_PER_TILE + f * K, K)])
        plsc.subcore_barrier()

        def stage_body(t, _):
            off = pl.multiple_of(chunk_base + t * STAGE, 8)
            pltpu.sync_copy(src_hbm.at[pl.ds(off, STAGE)], src_v)
            pltpu.sync_copy(dst_hbm.at[pl.ds(off, STAGE)], dst_v)

            def chunk(i, _):
                g0 = pltpu.async_copy(h_hbm.at[src_v.at[2 * i]], rows0_v, gsem)
                g1 = pltpu.async_copy(h_hbm.at[src_v.at[2 * i + 1]], rows1_v, gsem)
                g0.wait()
                s0 = pltpu.async_copy(rows0_v, agg_sh.at[dst_v.at[2 * i]],
                                      ssem, add=True)
                g1.wait()
                s1 = pltpu.async_copy(rows1_v, agg_sh.at[dst_v.at[2 * i + 1]],
                                      ssem, add=True)
                s0.wait()
                s1.wait()
                return 0
            lax.fori_loop(0, STAGE // 2, chunk, 0)
            return 0
        lax.fori_loop(0, nstages, stage_body, 0)
        plsc.subcore_barrier()

        pltpu.sync_copy(agg_sh.at[pl.ds(s * ROWS_PER_TILE, ROWS_PER_TILE)],
                        out_hbm.at[c, pl.ds(s * ROWS_PER_TILE, ROWS_PER_TILE)])

    return segsum


_segsum = _make_segsum()


ZCH = 8192
CPT = N_PAD * V // NS


def _make_count():
    mesh = plsc.VectorSubcoreMesh(core_axis_name="c", subcore_axis_name="s",
                                  num_cores=NC, num_subcores=NS)

    @functools.partial(
        pl.kernel,
        out_type=jax.ShapeDtypeStruct((NC, N_PAD * V), jnp.float32),
        mesh=mesh,
        scratch_types=[
            pltpu.VMEM((STAGE, K), jnp.int32),
            pltpu.VMEM((STAGE, K), jnp.int32),
            pltpu.VMEM((K,), jnp.int32),
            pltpu.VMEM((K,), jnp.int32),
            pltpu.VMEM((K,), jnp.float32),
            pltpu.VMEM((ZCH,), jnp.float32),
            pltpu.VMEM_SHARED((N_PAD * V,), jnp.float32),
        ],
    )
    def count(x_hbm, src_hbm, dst_hbm, out_hbm,
              src_v, dst_v, xvals_v, fidx_v, ones_v, zero_v, c_sh):
        c = lax.axis_index("c")
        s = lax.axis_index("s")
        wid = s * NC + c

        def zfill(i, _):
            zero_v[pl.ds(i * 16, 16)] = jnp.zeros((16,), jnp.float32)
            return 0
        lax.fori_loop(0, ZCH // 16, zfill, 0)
        for j in range(K // 16):
            ones_v[pl.ds(j * 16, 16)] = jnp.ones((16,), jnp.float32)
        for q in range(CPT // ZCH):
            pltpu.sync_copy(zero_v, c_sh.at[pl.ds(s * CPT + q * ZCH, ZCH)])
        plsc.subcore_barrier()

        def stage_body(t, _):
            pltpu.sync_copy(src_hbm.at[wid, pl.ds(t * STAGE, STAGE)], src_v)
            pltpu.sync_copy(dst_hbm.at[wid, pl.ds(t * STAGE, STAGE)], dst_v)

            def chunk(i, _):
                pltpu.sync_copy(x_hbm.at[src_v.at[i]], xvals_v)
                for j in range(K // 16):
                    dv = dst_v[i, pl.ds(j * 16, 16)]
                    xv = xvals_v[pl.ds(j * 16, 16)]
                    fidx_v[pl.ds(j * 16, 16)] = dv * V + xv
                pltpu.sync_copy(ones_v, c_sh.at[fidx_v], add=True)
                return 0
            lax.fori_loop(0, STAGE, chunk, 0)
            return 0
        lax.fori_loop(0, NSTAGE, stage_body, 0)
        plsc.subcore_barrier()

        pltpu.sync_copy(c_sh.at[pl.ds(s * CPT, CPT)],
                        out_hbm.at[c, pl.ds(s * CPT, CPT)])

    return count


_count = _make_count()


def _mlp0_body(x_ref, c_ref, embd_ref, w1_ref, b1_ref, w2_ref, b2_ref, out_ref):
    xb = x_ref[0, 0, :]
    iota = lax.broadcasted_iota(jnp.int32, (TB, V), 1)
    oh = (xb[:, None] == iota).astype(jnp.float32)
    q = oh + c_ref[0] + c_ref[1]
    z = jnp.dot(q, embd_ref[...], preferred_element_type=jnp.float32)
    z = jnp.dot(z, w1_ref[...], preferred_element_type=jnp.float32) + b1_ref[...]
    z = jnp.maximum(z, 0.0)
    z = jnp.dot(z, w2_ref[...], preferred_element_type=jnp.float32) + b2_ref[...]
    out_ref[...] = jnp.maximum(z, 0.0)


def _mlp0(x_r, counts, embd, w1, b1, w2, b2):
    return pl.pallas_call(
        _mlp0_body,
        grid=(NTB,),
        in_specs=[
            pl.BlockSpec((1, 1, TB), lambda i: (i, 0, 0)),
            pl.BlockSpec((NC, TB, V), lambda i: (0, i, 0)),
            pl.BlockSpec((V, D), lambda i: (0, 0)),
            pl.BlockSpec((D, D), lambda i: (0, 0)),
            pl.BlockSpec((1, D), lambda i: (0, 0)),
            pl.BlockSpec((D, D), lambda i: (0, 0)),
            pl.BlockSpec((1, D), lambda i: (0, 0)),
        ],
        out_specs=pl.BlockSpec((TB, D), lambda i: (i, 0)),
        out_shape=jax.ShapeDtypeStruct((N, D), jnp.float32),
    )(x_r, counts, embd, w1, b1, w2, b2)


def _embed_body(x_ref, embd_ref, out_ref):
    xb = x_ref[0, 0, :]
    iota = lax.broadcasted_iota(jnp.int32, (TB, V), 1)
    oh = (xb[:, None] == iota).astype(jnp.float32)
    out_ref[...] = jnp.dot(oh, embd_ref[...], preferred_element_type=jnp.float32)


def _embed(x_r, embd):
    return pl.pallas_call(
        _embed_body,
        grid=(NTB,),
        in_specs=[
            pl.BlockSpec((1, 1, TB), lambda i: (i, 0, 0)),
            pl.BlockSpec((V, D), lambda i: (0, 0)),
        ],
        out_specs=pl.BlockSpec((TB, D), lambda i: (i, 0)),
        out_shape=jax.ShapeDtypeStruct((N, D), jnp.float32),
    )(x_r, embd)


def _mlp_block(h_ref, agg_ref, w1_ref, b1_ref, w2_ref, b2_ref):
    z = h_ref[...] + agg_ref[0] + agg_ref[1]
    z = jnp.dot(z, w1_ref[...], preferred_element_type=jnp.float32) + b1_ref[...]
    z = jnp.maximum(z, 0.0)
    return jnp.dot(z, w2_ref[...], preferred_element_type=jnp.float32) + b2_ref[...]


def _mlp_body(h_ref, agg_ref, w1_ref, b1_ref, w2_ref, b2_ref, out_ref, *, relu_out):
    z = _mlp_block(h_ref, agg_ref, w1_ref, b1_ref, w2_ref, b2_ref)
    if relu_out:
        z = jnp.maximum(z, 0.0)
    out_ref[...] = z


def _mlp(h, agg, w1, b1, w2, b2, relu_out):
    return pl.pallas_call(
        functools.partial(_mlp_body, relu_out=relu_out),
        grid=(NTB,),
        in_specs=[
            pl.BlockSpec((TB, D), lambda i: (i, 0)),
            pl.BlockSpec((NC, TB, D), lambda i: (0, i, 0)),
            pl.BlockSpec((D, D), lambda i: (0, 0)),
            pl.BlockSpec((1, D), lambda i: (0, 0)),
            pl.BlockSpec((D, D), lambda i: (0, 0)),
            pl.BlockSpec((1, D), lambda i: (0, 0)),
        ],
        out_specs=pl.BlockSpec((TB, D), lambda i: (i, 0)),
        out_shape=jax.ShapeDtypeStruct((N, D), jnp.float32),
    )(h, agg, w1, b1, w2, b2)


def _final_body(h_ref, agg_ref, w1_ref, b1_ref, w2_ref, b2_ref,
                batch_ref, linw_ref, linb_ref, wpw_ref, wpb_ref,
                out_ref, sums_scr, counts_scr):
    i = pl.program_id(0)

    @pl.when(i == 0)
    def _init():
        sums_scr[...] = jnp.zeros((G, D), jnp.float32)
        counts_scr[...] = jnp.zeros((G, 1), jnp.float32)

    z = _mlp_block(h_ref, agg_ref, w1_ref, b1_ref, w2_ref, b2_ref)

    seg = batch_ref[0, 0, :]
    gio = lax.broadcasted_iota(jnp.int32, (G, TB), 0)
    p = (gio == seg[None, :]).astype(jnp.float32)
    sums_scr[...] += jnp.dot(p, z, preferred_element_type=jnp.float32)
    counts_scr[...] += jnp.dot(p, jnp.ones((TB, 1), jnp.float32),
                               preferred_element_type=jnp.float32)

    @pl.when(i == NTB - 1)
    def _head():
        m = sums_scr[...] / jnp.maximum(counts_scr[...], 1.0)
        lb = linb_ref[...]
        m = jnp.maximum(jnp.dot(m, linw_ref[0], preferred_element_type=jnp.float32)
                        + lb[0:1, :], 0.0)
        m = jnp.maximum(jnp.dot(m, linw_ref[1], preferred_element_type=jnp.float32)
                        + lb[1:2, :], 0.0)
        out_ref[...] = (jnp.dot(m, wpw_ref[...], preferred_element_type=jnp.float32)
                        + wpb_ref[...])

    del _init, _head


def _final(h, agg, w1, b1, w2, b2, batch_r, lin_W, lin_b, wp_W, wp_b):
    return pl.pallas_call(
        _final_body,
        grid=(NTB,),
        in_specs=[
            pl.BlockSpec((TB, D), lambda i: (i, 0)),
            pl.BlockSpec((NC, TB, D), lambda i: (0, i, 0)),
            pl.BlockSpec((D, D), lambda i: (0, 0)),
            pl.BlockSpec((1, D), lambda i: (0, 0)),
            pl.BlockSpec((D, D), lambda i: (0, 0)),
            pl.BlockSpec((1, D), lambda i: (0, 0)),
            pl.BlockSpec((1, 1, TB), lambda i: (i, 0, 0)),
            pl.BlockSpec((2, D, D), lambda i: (0, 0, 0)),
            pl.BlockSpec((2, D), lambda i: (0, 0)),
            pl.BlockSpec((D, 1), lambda i: (0, 0)),
            pl.BlockSpec((1, 1), lambda i: (0, 0)),
        ],
        out_specs=pl.BlockSpec((G, 1), lambda i: (0, 0)),
        out_shape=jax.ShapeDtypeStruct((G, 1), jnp.float32),
        scratch_shapes=[
            pltpu.VMEM((G, D), jnp.float32),
            pltpu.VMEM((G, 1), jnp.float32),
        ],
    )(h, agg, w1, b1, w2, b2, batch_r, lin_W, lin_b, wp_W, wp_b)


def kernel(x, edge_index, batch, embd, gin_W1, gin_b1, gin_W2, gin_b2,
           lin_W, lin_b, wp_W, wp_b):
    x_r = x.astype(jnp.int32).reshape(NTB, 1, TB)
    npad = E_PAD - E
    src_r = jnp.concatenate(
        [edge_index[0].astype(jnp.int32),
         jnp.zeros((npad,), jnp.int32)]).reshape(NW, NCHUNK, K)
    dst_r = jnp.concatenate(
        [edge_index[1].astype(jnp.int32),
         jnp.full((npad,), PAD_DST, jnp.int32)]).reshape(NW, NCHUNK, K)
    batch_r = batch.astype(jnp.int32).reshape(NTB, 1, TB)

    x_pad = jnp.concatenate(
        [x.astype(jnp.int32), jnp.zeros((N_PAD - N,), jnp.int32)])
    src_f = src_r.reshape(NCHUNKS_ALL, K)
    dst_f = dst_r.reshape(NCHUNKS_ALL, K)
    counts = _count(x_pad, src_r, dst_r).reshape(NC, N_PAD, V)
    h = _mlp0(x_r, counts, embd, gin_W1[0], gin_b1[0][None, :],
              gin_W2[0], gin_b2[0][None, :])
    agg = _segsum(h, src_f, dst_f)
    h = _mlp(h, agg, gin_W1[1], gin_b1[1][None, :],
             gin_W2[1], gin_b2[1][None, :], relu_out=True)
    agg = _segsum(h, src_f, dst_f)
    props = _final(h, agg, gin_W1[2], gin_b1[2][None, :],
                   gin_W2[2], gin_b2[2][None, :],
                   batch_r, lin_W, lin_b, wp_W, wp_b.reshape(1, 1))
    return props[:, 0]


# speedup vs baseline: 1.1753x; 1.1753x over previous
import functools

import jax
import jax.numpy as jnp
from jax import lax
from jax.experimental import pallas as pl
from jax.experimental.pallas import tpu as pltpu
from jax.experimental.pallas import tpu_sc as plsc

N = 10000
E = 320000
D = 128
V = 128
G = 256
NUM_LAYERS = 3

NC = 2
NS = 16
NW = NC * NS
K = 128
NCHUNK = 80
E_PAD = NW * NCHUNK * K
EW = E_PAD // NW
STAGE = 40
NSTAGE = NCHUNK // STAGE
NCHUNKS_ALL = E_PAD // K
C_CORE0 = 120
C_CORE1 = 40
OFF_CORE1 = NS * C_CORE0
N_PAD = 10240
ROWS_PER_TILE = N_PAD // NS
FLUSH_CHUNKS = ROWS_PER_TILE // K
PAD_DST = N + 8

TB = 400
NTB = N // TB


def _make_segsum():
    mesh = plsc.VectorSubcoreMesh(core_axis_name="c", subcore_axis_name="s",
                                  num_cores=NC, num_subcores=NS)

    @functools.partial(
        pl.kernel,
        out_type=jax.ShapeDtypeStruct((NC, N_PAD, D), jnp.float32),
        mesh=mesh,
        scratch_types=[
            pltpu.VMEM((STAGE, K), jnp.int32),
            pltpu.VMEM((STAGE, K), jnp.int32),
            pltpu.VMEM((K, D), jnp.float32),
            pltpu.VMEM((K, D), jnp.float32),
            pltpu.VMEM_SHARED((N_PAD, D), jnp.float32),
            pltpu.SemaphoreType.DMA,
            pltpu.SemaphoreType.DMA,
        ],
    )
    def segsum(h_hbm, src_hbm, dst_hbm, out_hbm,
               src_v, dst_v, rows0_v, rows1_v, agg_sh, gsem, ssem):
        c = lax.axis_index("c")
        s = lax.axis_index("s")
        chunk_base = jnp.where(c == 0, s * C_CORE0, OFF_CORE1 + s * C_CORE1)
        nstages = jnp.where(c == 0, C_CORE0 // STAGE, C_CORE1 // STAGE)

        def zrow(i, _):
            for j in range(D // 16):
                rows0_v[i, pl.ds(j * 16, 16)] = jnp.zeros((16,), jnp.float32)
            return 0
        lax.fori_loop(0, K, zrow, 0)
        for f in range(FLUSH_CHUNKS):
            pltpu.sync_copy(rows0_v, agg_sh.at[pl.ds(s * ROWS_PER_TILE + f * K, K)])
        plsc.subcore_barrier()

        def stage_body(t, _):
            off = pl.multiple_of(chunk_base + t * STAGE, 8)
            pltpu.sync_copy(src_hbm.at[pl.ds(off, STAGE)], src_v)
            pltpu.sync_copy(dst_hbm.at[pl.ds(off, STAGE)], dst_v)

            def chunk(i, _):
                g0 = pltpu.async_copy(h_hbm.at[src_v.at[2 * i]], rows0_v, gsem)
                g1 = pltpu.async_copy(h_hbm.at[src_v.at[2 * i + 1]], rows1_v, gsem)
                g0.wait()
                s0 = pltpu.async_copy(rows0_v, agg_sh.at[dst_v.at[2 * i]],
                                      ssem, add=True)
                g1.wait()
                s1 = pltpu.async_copy(rows1_v, agg_sh.at[dst_v.at[2 * i + 1]],
                                      ssem, add=True)
                s0.wait()
                s1.wait()
                return 0
            lax.fori_loop(0, STAGE // 2, chunk, 0)
            return 0
        lax.fori_loop(0, nstages, stage_body, 0)
        plsc.subcore_barrier()

        pltpu.sync_copy(agg_sh.at[pl.ds(s * ROWS_PER_TILE, ROWS_PER_TILE)],
                        out_hbm.at[c, pl.ds(s * ROWS_PER_TILE, ROWS_PER_TILE)])

    return segsum


_segsum = _make_segsum()


ZCH = 8192
CPT = N_PAD * V // NS


def _make_count():
    mesh = plsc.VectorSubcoreMesh(core_axis_name="c", subcore_axis_name="s",
                                  num_cores=NC, num_subcores=NS)

    @functools.partial(
        pl.kernel,
        out_type=jax.ShapeDtypeStruct((NC, N_PAD * V), jnp.float32),
        mesh=mesh,
        scratch_types=[
            pltpu.VMEM((STAGE, K), jnp.int32),
            pltpu.VMEM((STAGE, K), jnp.int32),
            pltpu.VMEM((K,), jnp.int32),
            pltpu.VMEM((K,), jnp.int32),
            pltpu.VMEM((K,), jnp.float32),
            pltpu.VMEM((ZCH,), jnp.float32),
            pltpu.VMEM_SHARED((N_PAD * V,), jnp.float32),
        ],
    )
    def count(x_hbm, src_hbm, dst_hbm, out_hbm,
              src_v, dst_v, xvals_v, fidx_v, ones_v, zero_v, c_sh):
        c = lax.axis_index("c")
        s = lax.axis_index("s")
        wid = s * NC + c

        def zfill(i, _):
            zero_v[pl.ds(i * 16, 16)] = jnp.zeros((16,), jnp.float32)
            return 0
        lax.fori_loop(0, ZCH // 16, zfill, 0)
        for j in range(K // 16):
            ones_v[pl.ds(j * 16, 16)] = jnp.ones((16,), jnp.float32)
        for q in range(CPT // ZCH):
            pltpu.sync_copy(zero_v, c_sh.at[pl.ds(s * CPT + q * ZCH, ZCH)])
        plsc.subcore_barrier()

        def stage_body(t, _):
            pltpu.sync_copy(src_hbm.at[wid, pl.ds(t * STAGE, STAGE)], src_v)
            pltpu.sync_copy(dst_hbm.at[wid, pl.ds(t * STAGE, STAGE)], dst_v)

            def chunk(i, _):
                pltpu.sync_copy(x_hbm.at[src_v.at[i]], xvals_v)
                for j in range(K // 16):
                    dv = dst_v[i, pl.ds(j * 16, 16)]
                    xv = xvals_v[pl.ds(j * 16, 16)]
                    fidx_v[pl.ds(j * 16, 16)] = dv * V + xv
                pltpu.sync_copy(ones_v, c_sh.at[fidx_v], add=True)
                return 0
            lax.fori_loop(0, STAGE, chunk, 0)
            return 0
        lax.fori_loop(0, NSTAGE, stage_body, 0)
        plsc.subcore_barrier()

        pltpu.sync_copy(c_sh.at[pl.ds(s * CPT, CPT)],
                        out_hbm.at[c, pl.ds(s * CPT, CPT)])

    return count


_count = _make_count()


def _mlp0_body(x_ref, c_ref, embd_ref, w1_ref, b1_ref, w2_ref, b2_ref, out_ref):
    xb = x_ref[0, 0, :]
    iota = lax.broadcasted_iota(jnp.int32, (TB, V), 1)
    oh = (xb[:, None] == iota).astype(jnp.float32)
    q = oh + c_ref[0] + c_ref[1]
    z = jnp.dot(q, embd_ref[...], preferred_element_type=jnp.float32)
    z = jnp.dot(z, w1_ref[...], preferred_element_type=jnp.float32) + b1_ref[...]
    z = jnp.maximum(z, 0.0)
    z = jnp.dot(z, w2_ref[...], preferred_element_type=jnp.float32) + b2_ref[...]
    out_ref[...] = jnp.maximum(z, 0.0)


def _mlp0(x_r, counts, embd, w1, b1, w2, b2):
    return pl.pallas_call(
        _mlp0_body,
        grid=(NTB,),
        in_specs=[
            pl.BlockSpec((1, 1, TB), lambda i: (i, 0, 0)),
            pl.BlockSpec((NC, TB, V), lambda i: (0, i, 0)),
            pl.BlockSpec((V, D), lambda i: (0, 0)),
            pl.BlockSpec((D, D), lambda i: (0, 0)),
            pl.BlockSpec((1, D), lambda i: (0, 0)),
            pl.BlockSpec((D, D), lambda i: (0, 0)),
            pl.BlockSpec((1, D), lambda i: (0, 0)),
        ],
        out_specs=pl.BlockSpec((TB, D), lambda i: (i, 0)),
        out_shape=jax.ShapeDtypeStruct((N, D), jnp.float32),
    )(x_r, counts, embd, w1, b1, w2, b2)


def _embed_body(x_ref, embd_ref, out_ref):
    xb = x_ref[0, 0, :]
    iota = lax.broadcasted_iota(jnp.int32, (TB, V), 1)
    oh = (xb[:, None] == iota).astype(jnp.float32)
    out_ref[...] = jnp.dot(oh, embd_ref[...], preferred_element_type=jnp.float32)


def _embed(x_r, embd):
    return pl.pallas_call(
        _embed_body,
        grid=(NTB,),
        in_specs=[
            pl.BlockSpec((1, 1, TB), lambda i: (i, 0, 0)),
            pl.BlockSpec((V, D), lambda i: (0, 0)),
        ],
        out_specs=pl.BlockSpec((TB, D), lambda i: (i, 0)),
        out_shape=jax.ShapeDtypeStruct((N, D), jnp.float32),
    )(x_r, embd)


def _mlp_block(h_ref, agg_ref, w1_ref, b1_ref, w2_ref, b2_ref):
    z = h_ref[...] + agg_ref[0] + agg_ref[1]
    z = jnp.dot(z, w1_ref[...], preferred_element_type=jnp.float32) + b1_ref[...]
    z = jnp.maximum(z, 0.0)
    return jnp.dot(z, w2_ref[...], preferred_element_type=jnp.float32) + b2_ref[...]


def _mlp_body(h_ref, agg_ref, w1_ref, b1_ref, w2_ref, b2_ref, out_ref, *, relu_out):
    z = _mlp_block(h_ref, agg_ref, w1_ref, b1_ref, w2_ref, b2_ref)
    if relu_out:
        z = jnp.maximum(z, 0.0)
    out_ref[...] = z


def _mlp(h, agg, w1, b1, w2, b2, relu_out):
    return pl.pallas_call(
        functools.partial(_mlp_body, relu_out=relu_out),
        grid=(NTB,),
        in_specs=[
            pl.BlockSpec((TB, D), lambda i: (i, 0)),
            pl.BlockSpec((NC, TB, D), lambda i: (0, i, 0)),
            pl.BlockSpec((D, D), lambda i: (0, 0)),
            pl.BlockSpec((1, D), lambda i: (0, 0)),
            pl.BlockSpec((D, D), lambda i: (0, 0)),
            pl.BlockSpec((1, D), lambda i: (0, 0)),
        ],
        out_specs=pl.BlockSpec((TB, D), lambda i: (i, 0)),
        out_shape=jax.ShapeDtypeStruct((N, D), jnp.float32),
    )(h, agg, w1, b1, w2, b2)


def _final_body(h_ref, agg_ref, w1_ref, b1_ref, w2_ref, b2_ref,
                batch_ref, linw_ref, linb_ref, wpw_ref, wpb_ref,
                out_ref, sums_scr, counts_scr):
    i = pl.program_id(0)

    @pl.when(i == 0)
    def _init():
        sums_scr[...] = jnp.zeros((G, D), jnp.float32)
        counts_scr[...] = jnp.zeros((G, 1), jnp.float32)

    z = _mlp_block(h_ref, agg_ref, w1_ref, b1_ref, w2_ref, b2_ref)

    seg = batch_ref[0, 0, :]
    gio = lax.broadcasted_iota(jnp.int32, (G, TB), 0)
    p = (gio == seg[None, :]).astype(jnp.float32)
    sums_scr[...] += jnp.dot(p, z, preferred_element_type=jnp.float32)
    counts_scr[...] += jnp.dot(p, jnp.ones((TB, 1), jnp.float32),
                               preferred_element_type=jnp.float32)

    @pl.when(i == NTB - 1)
    def _head():
        m = sums_scr[...] / jnp.maximum(counts_scr[...], 1.0)
        lb = linb_ref[...]
        m = jnp.maximum(jnp.dot(m, linw_ref[0], preferred_element_type=jnp.float32)
                        + lb[0:1, :], 0.0)
        m = jnp.maximum(jnp.dot(m, linw_ref[1], preferred_element_type=jnp.float32)
                        + lb[1:2, :], 0.0)
        out_ref[...] = (jnp.dot(m, wpw_ref[...], preferred_element_type=jnp.float32)
                        + wpb_ref[...])

    del _init, _head


def _final(h, agg, w1, b1, w2, b2, batch_r, lin_W, lin_b, wp_W, wp_b):
    return pl.pallas_call(
        _final_body,
        grid=(NTB,),
        in_specs=[
            pl.BlockSpec((TB, D), lambda i: (i, 0)),
            pl.BlockSpec((NC, TB, D), lambda i: (0, i, 0)),
            pl.BlockSpec((D, D), lambda i: (0, 0)),
            pl.BlockSpec((1, D), lambda i: (0, 0)),
            pl.BlockSpec((D, D), lambda i: (0, 0)),
            pl.BlockSpec((1, D), lambda i: (0, 0)),
            pl.BlockSpec((1, 1, TB), lambda i: (i, 0, 0)),
            pl.BlockSpec((2, D, D), lambda i: (0, 0, 0)),
            pl.BlockSpec((2, D), lambda i: (0, 0)),
            pl.BlockSpec((D, 1), lambda i: (0, 0)),
            pl.BlockSpec((1, 1), lambda i: (0, 0)),
        ],
        out_specs=pl.BlockSpec((G, 1), lambda i: (0, 0)),
        out_shape=jax.ShapeDtypeStruct((G, 1), jnp.float32),
        scratch_shapes=[
            pltpu.VMEM((G, D), jnp.float32),
            pltpu.VMEM((G, 1), jnp.float32),
        ],
    )(h, agg, w1, b1, w2, b2, batch_r, lin_W, lin_b, wp_W, wp_b)


def kernel(x, edge_index, batch, embd, gin_W1, gin_b1, gin_W2, gin_b2,
           lin_W, lin_b, wp_W, wp_b):
    x_r = x.astype(jnp.int32).reshape(NTB, 1, TB)
    npad = E_PAD - E
    src_r = jnp.concatenate(
        [edge_index[0].astype(jnp.int32),
         jnp.zeros((npad,), jnp.int32)]).reshape(NW, NCHUNK, K)
    dst_r = jnp.concatenate(
        [edge_index[1].astype(jnp.int32),
         jnp.full((npad,), PAD_DST, jnp.int32)]).reshape(NW, NCHUNK, K)
    batch_r = batch.astype(jnp.int32).reshape(NTB, 1, TB)

    x_pad = jnp.concatenate(
        [x.astype(jnp.int32), jnp.zeros((N_PAD - N,), jnp.int32)])
    src_f = src_r.reshape(NCHUNKS_ALL, K)
    dst_f = dst_r.reshape(NCHUNKS_ALL, K)
    counts = _count(x_pad, src_r, dst_r).reshape(NC, N_PAD, V)
    h = _mlp0(x_r, counts, embd, gin_W1[0], gin_b1[0][None, :],
              gin_W2[0], gin_b2[0][None, :])
    agg = _segsum(h, src_f, dst_f)
    h = _mlp(h, agg, gin_W1[1], gin_b1[1][None, :],
             gin_W2[1], gin_b2[1][None, :], relu_out=True)
    agg = _segsum(h, src_f, dst_f)
    props = _final(h, agg, gin_W1[2], gin_b1[2][None, :],
                   gin_W2[2], gin_b2[2][None, :],
                   batch_r, lin_W, lin_b, wp_W, wp_b.reshape(1, 1))
    return props[:, 0]


# speedup vs baseline: 1.2309x; 1.0473x over previous
import functools

import jax
import jax.numpy as jnp
from jax import lax
from jax.experimental import pallas as pl
from jax.experimental.pallas import tpu as pltpu
from jax.experimental.pallas import tpu_sc as plsc

N = 10000
E = 320000
D = 128
V = 128
G = 256
NUM_LAYERS = 3

NC = 2
NS = 16
NW = NC * NS
K = 128
NCHUNK = 80
E_PAD = NW * NCHUNK * K
EW = E_PAD // NW
STAGE = 40
NSTAGE = NCHUNK // STAGE
KG = 80
NCHUNKS_ALL = E_PAD // KG
C_CORE0 = 128
C_CORE1 = 128
OFF_CORE1 = NS * C_CORE0
GSTAGE = 32
NBUF = 4
N_PAD = 10240
ROWS_PER_TILE = N_PAD // NS
FLUSH_CHUNKS = ROWS_PER_TILE // K
PAD_DST = N + 8

TB = 400
NTB = N // TB


def _make_segsum():
    mesh = plsc.VectorSubcoreMesh(core_axis_name="c", subcore_axis_name="s",
                                  num_cores=NC, num_subcores=NS)

    @functools.partial(
        pl.kernel,
        out_type=jax.ShapeDtypeStruct((NC, N_PAD, D), jnp.float32),
        mesh=mesh,
        scratch_types=[
            pltpu.VMEM((GSTAGE, KG), jnp.int32),
            pltpu.VMEM((GSTAGE, KG), jnp.int32),
            [pltpu.VMEM((KG, D), jnp.float32) for _ in range(NBUF)],
            pltpu.VMEM_SHARED((N_PAD, D), jnp.float32),
            pltpu.SemaphoreType.DMA,
            pltpu.SemaphoreType.DMA,
        ],
    )
    def segsum(h_hbm, src_hbm, dst_hbm, out_hbm,
               src_v, dst_v, rows, agg_sh, gsem, ssem):
        c = lax.axis_index("c")
        s = lax.axis_index("s")
        chunk_base = (c * NS + s) * C_CORE0

        def zrow(i, _):
            for j in range(D // 16):
                rows[0][i, pl.ds(j * 16, 16)] = jnp.zeros((16,), jnp.float32)
            return 0
        lax.fori_loop(0, KG, zrow, 0)
        for f in range(ROWS_PER_TILE // KG):
            pltpu.sync_copy(rows[0],
                            agg_sh.at[pl.ds(s * ROWS_PER_TILE + f * KG, KG)])
        plsc.subcore_barrier()

        def stage_body(t, _):
            off = pl.multiple_of(chunk_base + t * GSTAGE, 8)
            pltpu.sync_copy(src_hbm.at[pl.ds(off, GSTAGE)], src_v)
            pltpu.sync_copy(dst_hbm.at[pl.ds(off, GSTAGE)], dst_v)

            def chunk(i, _):
                gs = [pltpu.async_copy(h_hbm.at[src_v.at[NBUF * i + b]],
                                       rows[b], gsem)
                      for b in range(NBUF)]
                ss = []
                for b in range(NBUF):
                    gs[b].wait()
                    ss.append(pltpu.async_copy(
                        rows[b], agg_sh.at[dst_v.at[NBUF * i + b]],
                        ssem, add=True))
                for b in range(NBUF):
                    ss[b].wait()
                return 0
            lax.fori_loop(0, GSTAGE // NBUF, chunk, 0)
            return 0
        lax.fori_loop(0, C_CORE0 // GSTAGE, stage_body, 0)
        plsc.subcore_barrier()

        pltpu.sync_copy(agg_sh.at[pl.ds(s * ROWS_PER_TILE, ROWS_PER_TILE)],
                        out_hbm.at[c, pl.ds(s * ROWS_PER_TILE, ROWS_PER_TILE)])

    return segsum


_segsum = _make_segsum()


ZCH = 8192
CPT = N_PAD * V // NS


def _make_count():
    mesh = plsc.VectorSubcoreMesh(core_axis_name="c", subcore_axis_name="s",
                                  num_cores=NC, num_subcores=NS)

    @functools.partial(
        pl.kernel,
        out_type=jax.ShapeDtypeStruct((NC, N_PAD * V), jnp.float32),
        mesh=mesh,
        scratch_types=[
            pltpu.VMEM((STAGE, K), jnp.int32),
            pltpu.VMEM((STAGE, K), jnp.int32),
            pltpu.VMEM((K,), jnp.int32),
            pltpu.VMEM((K,), jnp.int32),
            pltpu.VMEM((K,), jnp.float32),
            pltpu.VMEM((ZCH,), jnp.float32),
            pltpu.VMEM_SHARED((N_PAD * V,), jnp.float32),
        ],
    )
    def count(x_hbm, src_hbm, dst_hbm, out_hbm,
              src_v, dst_v, xvals_v, fidx_v, ones_v, zero_v, c_sh):
        c = lax.axis_index("c")
        s = lax.axis_index("s")
        wid = s * NC + c

        def zfill(i, _):
            zero_v[pl.ds(i * 16, 16)] = jnp.zeros((16,), jnp.float32)
            return 0
        lax.fori_loop(0, ZCH // 16, zfill, 0)
        for j in range(K // 16):
            ones_v[pl.ds(j * 16, 16)] = jnp.ones((16,), jnp.float32)
        for q in range(CPT // ZCH):
            pltpu.sync_copy(zero_v, c_sh.at[pl.ds(s * CPT + q * ZCH, ZCH)])
        plsc.subcore_barrier()

        def stage_body(t, _):
            pltpu.sync_copy(src_hbm.at[wid, pl.ds(t * STAGE, STAGE)], src_v)
            pltpu.sync_copy(dst_hbm.at[wid, pl.ds(t * STAGE, STAGE)], dst_v)

            def chunk(i, _):
                pltpu.sync_copy(x_hbm.at[src_v.at[i]], xvals_v)
                for j in range(K // 16):
                    dv = dst_v[i, pl.ds(j * 16, 16)]
                    xv = xvals_v[pl.ds(j * 16, 16)]
                    fidx_v[pl.ds(j * 16, 16)] = dv * V + xv
                pltpu.sync_copy(ones_v, c_sh.at[fidx_v], add=True)
                return 0
            lax.fori_loop(0, STAGE, chunk, 0)
            return 0
        lax.fori_loop(0, NSTAGE, stage_body, 0)
        plsc.subcore_barrier()

        pltpu.sync_copy(c_sh.at[pl.ds(s * CPT, CPT)],
                        out_hbm.at[c, pl.ds(s * CPT, CPT)])

    return count


_count = _make_count()


def _mlp0_body(x_ref, c_ref, embd_ref, w1_ref, b1_ref, w2_ref, b2_ref, out_ref):
    xb = x_ref[0, 0, :]
    iota = lax.broadcasted_iota(jnp.int32, (TB, V), 1)
    oh = (xb[:, None] == iota).astype(jnp.float32)
    q = oh + c_ref[0] + c_ref[1]
    z = jnp.dot(q, embd_ref[...], preferred_element_type=jnp.float32)
    z = jnp.dot(z, w1_ref[...], preferred_element_type=jnp.float32) + b1_ref[...]
    z = jnp.maximum(z, 0.0)
    z = jnp.dot(z, w2_ref[...], preferred_element_type=jnp.float32) + b2_ref[...]
    out_ref[...] = jnp.maximum(z, 0.0)


def _mlp0(x_r, counts, embd, w1, b1, w2, b2):
    return pl.pallas_call(
        _mlp0_body,
        grid=(NTB,),
        in_specs=[
            pl.BlockSpec((1, 1, TB), lambda i: (i, 0, 0)),
            pl.BlockSpec((NC, TB, V), lambda i: (0, i, 0)),
            pl.BlockSpec((V, D), lambda i: (0, 0)),
            pl.BlockSpec((D, D), lambda i: (0, 0)),
            pl.BlockSpec((1, D), lambda i: (0, 0)),
            pl.BlockSpec((D, D), lambda i: (0, 0)),
            pl.BlockSpec((1, D), lambda i: (0, 0)),
        ],
        out_specs=pl.BlockSpec((TB, D), lambda i: (i, 0)),
        out_shape=jax.ShapeDtypeStruct((N, D), jnp.float32),
    )(x_r, counts, embd, w1, b1, w2, b2)


def _embed_body(x_ref, embd_ref, out_ref):
    xb = x_ref[0, 0, :]
    iota = lax.broadcasted_iota(jnp.int32, (TB, V), 1)
    oh = (xb[:, None] == iota).astype(jnp.float32)
    out_ref[...] = jnp.dot(oh, embd_ref[...], preferred_element_type=jnp.float32)


def _embed(x_r, embd):
    return pl.pallas_call(
        _embed_body,
        grid=(NTB,),
        in_specs=[
            pl.BlockSpec((1, 1, TB), lambda i: (i, 0, 0)),
            pl.BlockSpec((V, D), lambda i: (0, 0)),
        ],
        out_specs=pl.BlockSpec((TB, D), lambda i: (i, 0)),
        out_shape=jax.ShapeDtypeStruct((N, D), jnp.float32),
    )(x_r, embd)


def _mlp_block(h_ref, agg_ref, w1_ref, b1_ref, w2_ref, b2_ref):
    z = h_ref[...] + agg_ref[0] + agg_ref[1]
    z = jnp.dot(z, w1_ref[...], preferred_element_type=jnp.float32) + b1_ref[...]
    z = jnp.maximum(z, 0.0)
    return jnp.dot(z, w2_ref[...], preferred_element_type=jnp.float32) + b2_ref[...]


def _mlp_body(h_ref, agg_ref, w1_ref, b1_ref, w2_ref, b2_ref, out_ref, *, relu_out):
    z = _mlp_block(h_ref, agg_ref, w1_ref, b1_ref, w2_ref, b2_ref)
    if relu_out:
        z = jnp.maximum(z, 0.0)
    out_ref[...] = z


def _mlp(h, agg, w1, b1, w2, b2, relu_out):
    return pl.pallas_call(
        functools.partial(_mlp_body, relu_out=relu_out),
        grid=(NTB,),
        in_specs=[
            pl.BlockSpec((TB, D), lambda i: (i, 0)),
            pl.BlockSpec((NC, TB, D), lambda i: (0, i, 0)),
            pl.BlockSpec((D, D), lambda i: (0, 0)),
            pl.BlockSpec((1, D), lambda i: (0, 0)),
            pl.BlockSpec((D, D), lambda i: (0, 0)),
            pl.BlockSpec((1, D), lambda i: (0, 0)),
        ],
        out_specs=pl.BlockSpec((TB, D), lambda i: (i, 0)),
        out_shape=jax.ShapeDtypeStruct((N, D), jnp.float32),
    )(h, agg, w1, b1, w2, b2)


def _final_body(h_ref, agg_ref, w1_ref, b1_ref, w2_ref, b2_ref,
                batch_ref, linw_ref, linb_ref, wpw_ref, wpb_ref,
                out_ref, sums_scr, counts_scr):
    i = pl.program_id(0)

    @pl.when(i == 0)
    def _init():
        sums_scr[...] = jnp.zeros((G, D), jnp.float32)
        counts_scr[...] = jnp.zeros((G, 1), jnp.float32)

    z = _mlp_block(h_ref, agg_ref, w1_ref, b1_ref, w2_ref, b2_ref)

    seg = batch_ref[0, 0, :]
    gio = lax.broadcasted_iota(jnp.int32, (G, TB), 0)
    p = (gio == seg[None, :]).astype(jnp.float32)
    sums_scr[...] += jnp.dot(p, z, preferred_element_type=jnp.float32)
    counts_scr[...] += jnp.dot(p, jnp.ones((TB, 1), jnp.float32),
                               preferred_element_type=jnp.float32)

    @pl.when(i == NTB - 1)
    def _head():
        m = sums_scr[...] / jnp.maximum(counts_scr[...], 1.0)
        lb = linb_ref[...]
        m = jnp.maximum(jnp.dot(m, linw_ref[0], preferred_element_type=jnp.float32)
                        + lb[0:1, :], 0.0)
        m = jnp.maximum(jnp.dot(m, linw_ref[1], preferred_element_type=jnp.float32)
                        + lb[1:2, :], 0.0)
        out_ref[...] = (jnp.dot(m, wpw_ref[...], preferred_element_type=jnp.float32)
                        + wpb_ref[...])

    del _init, _head


def _final(h, agg, w1, b1, w2, b2, batch_r, lin_W, lin_b, wp_W, wp_b):
    return pl.pallas_call(
        _final_body,
        grid=(NTB,),
        in_specs=[
            pl.BlockSpec((TB, D), lambda i: (i, 0)),
            pl.BlockSpec((NC, TB, D), lambda i: (0, i, 0)),
            pl.BlockSpec((D, D), lambda i: (0, 0)),
            pl.BlockSpec((1, D), lambda i: (0, 0)),
            pl.BlockSpec((D, D), lambda i: (0, 0)),
            pl.BlockSpec((1, D), lambda i: (0, 0)),
            pl.BlockSpec((1, 1, TB), lambda i: (i, 0, 0)),
            pl.BlockSpec((2, D, D), lambda i: (0, 0, 0)),
            pl.BlockSpec((2, D), lambda i: (0, 0)),
            pl.BlockSpec((D, 1), lambda i: (0, 0)),
            pl.BlockSpec((1, 1), lambda i: (0, 0)),
        ],
        out_specs=pl.BlockSpec((G, 1), lambda i: (0, 0)),
        out_shape=jax.ShapeDtypeStruct((G, 1), jnp.float32),
        scratch_shapes=[
            pltpu.VMEM((G, D), jnp.float32),
            pltpu.VMEM((G, 1), jnp.float32),
        ],
    )(h, agg, w1, b1, w2, b2, batch_r, lin_W, lin_b, wp_W, wp_b)


def kernel(x, edge_index, batch, embd, gin_W1, gin_b1, gin_W2, gin_b2,
           lin_W, lin_b, wp_W, wp_b):
    x_r = x.astype(jnp.int32).reshape(NTB, 1, TB)
    npad = E_PAD - E
    src_r = jnp.concatenate(
        [edge_index[0].astype(jnp.int32),
         jnp.zeros((npad,), jnp.int32)]).reshape(NW, NCHUNK, K)
    dst_r = jnp.concatenate(
        [edge_index[1].astype(jnp.int32),
         jnp.full((npad,), PAD_DST, jnp.int32)]).reshape(NW, NCHUNK, K)
    batch_r = batch.astype(jnp.int32).reshape(NTB, 1, TB)

    x_pad = jnp.concatenate(
        [x.astype(jnp.int32), jnp.zeros((N_PAD - N,), jnp.int32)])
    src_f = src_r.reshape(NCHUNKS_ALL, KG)
    dst_f = dst_r.reshape(NCHUNKS_ALL, KG)
    counts = _count(x_pad, src_r, dst_r).reshape(NC, N_PAD, V)
    h = _mlp0(x_r, counts, embd, gin_W1[0], gin_b1[0][None, :],
              gin_W2[0], gin_b2[0][None, :])
    agg = _segsum(h, src_f, dst_f)
    h = _mlp(h, agg, gin_W1[1], gin_b1[1][None, :],
             gin_W2[1], gin_b2[1][None, :], relu_out=True)
    agg = _segsum(h, src_f, dst_f)
    props = _final(h, agg, gin_W1[2], gin_b1[2][None, :],
                   gin_W2[2], gin_b2[2][None, :],
                   batch_r, lin_W, lin_b, wp_W, wp_b.reshape(1, 1))
    return props[:, 0]
